# baseline (speedup 1.0000x reference)
import functools

import jax
import jax.numpy as jnp
from jax import lax
from jax.experimental import pallas as pl
from jax.experimental.pallas import tpu as pltpu
from jax.experimental.pallas import tpu_sc as plsc

VOCAB = 100000
EMBED_DIM = 300
BATCH = 16384
TAIL = EMBED_DIM - 256


@functools.lru_cache(maxsize=None)
def _make_gather_kernel():
    info = plsc.get_sparse_core_info()
    nc, ns = info.num_cores, info.num_subcores
    nw = nc * ns
    bpw = BATCH // nw
    chunk = 64
    nch = bpw // chunk
    mesh = plsc.VectorSubcoreMesh(core_axis_name="c", subcore_axis_name="s")
    params = pltpu.CompilerParams(needs_layout_passes=False)

    @functools.partial(
        pl.kernel,
        mesh=mesh,
        compiler_params=params,
        out_type=[jax.ShapeDtypeStruct((BATCH, EMBED_DIM), jnp.float32)] * 3,
        scratch_types=[
            pltpu.VMEM((bpw,), jnp.int32),
            pltpu.VMEM((chunk, EMBED_DIM), jnp.float32),
            pltpu.VMEM((chunk, EMBED_DIM), jnp.float32),
            pltpu.VMEM((chunk, 128), jnp.float32),
            pltpu.VMEM((chunk, 128), jnp.float32),
            pltpu.SemaphoreType.DMA,
            pltpu.SemaphoreType.DMA,
            pltpu.SemaphoreType.DMA,
            pltpu.SemaphoreType.DMA,
        ],
    )
    def gather_k(table, tailp, tw, sy, an, o0, o1, o2,
                 idx_v, st0, st1, tb0, tb1, g0, g1, w0, w1):
        idx_in = (tw, sy, an)
        outs = (o0, o1, o2)
        stages = (st0, st1)
        tails = (tb0, tb1)
        gsems = (g0, g1)
        wsems = (w0, w1)
        wid = lax.axis_index("s") * nc + lax.axis_index("c")
        base = wid * bpw

        def pieces(c, p):
            isl = idx_v.at[pl.ds(c * chunk, chunk)]
            st = stages[p]
            return (
                (table.at[isl, pl.ds(0, 128)], st.at[:, pl.ds(0, 128)]),
                (table.at[isl, pl.ds(128, 128)], st.at[:, pl.ds(128, 128)]),
                (tailp.at[isl, pl.ds(0, 128)], tails[p]),
            )

        def prep(c, p):
            for src, dst in pieces(c, p):
                pltpu.async_copy(src, dst, gsems[p])

        def wait_gather(c, p):
            for src, dst in pieces(c, p):
                pltpu.make_async_copy(src, dst, gsems[p]).wait()

        def wait_write(out, p):
            pltpu.make_async_copy(
                stages[p], out.at[pl.ds(base, chunk)], wsems[p]).wait()

        def tail_fix(p):
            st = stages[p]
            tb = tails[p]

            @pl.loop(0, chunk)
            def row(r):
                for c in (0, 16, TAIL - 16):
                    st[r, pl.ds(256 + c, 16)] = tb[r, pl.ds(c, 16)]

        for a in range(3):
            out = outs[a]
            pltpu.sync_copy(idx_in[a].at[pl.ds(base, bpw)], idx_v)
            prep(0, 0)

            @pl.loop(0, nch, step=2)
            def chunk_pair(c0):
                for p in range(2):
                    c = c0 + p
                    nxt = c + 1
                    q = (p + 1) % 2

                    @pl.when(nxt < nch)
                    def _():
                        @pl.when(nxt >= 2)
                        def _():
                            wait_write(out, q)

                        prep(nxt, q)

                    wait_gather(c, p)
                    tail_fix(p)
                    pltpu.async_copy(
                        stages[p],
                        out.at[pl.ds(base + c * chunk, chunk)],
                        wsems[p])

            wait_write(out, 0)
            wait_write(out, 1)

    return gather_k


def kernel(table, target_word, synonym, antonym):
    tailp = jnp.pad(table[:, 256:], ((0, 0), (0, 128 - TAIL)))
    out = _make_gather_kernel()(table, tailp, target_word, synonym, antonym)
    return (out[0], out[1], out[2])


# speedup vs baseline: 1.0567x; 1.0567x over previous
import functools

import jax
import jax.numpy as jnp
from jax import lax
from jax.experimental import pallas as pl
from jax.experimental.pallas import tpu as pltpu
from jax.experimental.pallas import tpu_sc as plsc

VOCAB = 100000
EMBED_DIM = 300
BATCH = 16384
TAIL = EMBED_DIM - 256


@functools.lru_cache(maxsize=None)
def _make_gather_kernel():
    info = plsc.get_sparse_core_info()
    nc, ns = info.num_cores, info.num_subcores
    nw = nc * ns
    bpw = BATCH // nw
    chunk = 64
    nch = bpw // chunk
    mesh = plsc.VectorSubcoreMesh(core_axis_name="c", subcore_axis_name="s")
    params = pltpu.CompilerParams(needs_layout_passes=False)

    @functools.partial(
        pl.kernel,
        mesh=mesh,
        compiler_params=params,
        out_type=[jax.ShapeDtypeStruct((BATCH, EMBED_DIM), jnp.float32)] * 3,
        scratch_types=[
            pltpu.VMEM((bpw,), jnp.int32),
            pltpu.VMEM((chunk, EMBED_DIM), jnp.float32),
            pltpu.VMEM((chunk, EMBED_DIM), jnp.float32),
            pltpu.VMEM((chunk, 128), jnp.float32),
            pltpu.VMEM((chunk, 128), jnp.float32),
            pltpu.SemaphoreType.DMA,
            pltpu.SemaphoreType.DMA,
            pltpu.SemaphoreType.DMA,
            pltpu.SemaphoreType.DMA,
        ],
    )
    def gather_k(table, tailp, tw, sy, an, o0, o1, o2,
                 idx_v, st0, st1, tb0, tb1, g0, g1, w0, w1):
        idx_in = (tw, sy, an)
        outs = (o0, o1, o2)
        stages = (st0, st1)
        tails = (tb0, tb1)
        gsems = (g0, g1)
        wsems = (w0, w1)
        wid = lax.axis_index("s") * nc + lax.axis_index("c")
        base = wid * bpw

        def pieces(c, p):
            isl = idx_v.at[pl.ds(c * chunk, chunk)]
            st = stages[p]
            return (
                (table.at[isl, pl.ds(0, 128)], st.at[:, pl.ds(0, 128)]),
                (table.at[isl, pl.ds(128, 128)], st.at[:, pl.ds(128, 128)]),
                (tailp.at[isl, pl.ds(0, 128)], tails[p]),
            )

        def prep(c, p):
            for src, dst in pieces(c, p):
                pltpu.async_copy(src, dst, gsems[p])

        def wait_gather(c, p):
            for src, dst in pieces(c, p):
                pltpu.make_async_copy(src, dst, gsems[p]).wait()

        def wait_write(out, p):
            pltpu.make_async_copy(
                stages[p], out.at[pl.ds(base, chunk)], wsems[p]).wait()

        def tail_fix(p):
            st = stages[p]
            tb = tails[p]

            @pl.loop(0, chunk)
            def row(r):
                for c in (0, 16, TAIL - 16):
                    st[r, pl.ds(256 + c, 16)] = tb[r, pl.ds(84 + c, 16)]

        for a in range(3):
            out = outs[a]
            pltpu.sync_copy(idx_in[a].at[pl.ds(base, bpw)], idx_v)
            prep(0, 0)

            @pl.loop(0, nch, step=2)
            def chunk_pair(c0):
                for p in range(2):
                    c = c0 + p
                    nxt = c + 1
                    q = (p + 1) % 2

                    @pl.when(nxt < nch)
                    def _():
                        @pl.when(nxt >= 2)
                        def _():
                            wait_write(out, q)

                        prep(nxt, q)

                    wait_gather(c, p)
                    tail_fix(p)
                    pltpu.async_copy(
                        stages[p],
                        out.at[pl.ds(base + c * chunk, chunk)],
                        wsems[p])

            wait_write(out, 0)
            wait_write(out, 1)

    return gather_k


def kernel(table, target_word, synonym, antonym):
    tailp = table[:, EMBED_DIM - 128:]
    out = _make_gather_kernel()(table, tailp, target_word, synonym, antonym)
    return (out[0], out[1], out[2])
